# baseline (speedup 1.0000x reference)
import functools

import jax
import jax.numpy as jnp
from jax import lax
from jax.experimental import pallas as pl
from jax.experimental.pallas import tpu as pltpu
from jax.experimental.pallas import tpu_sc as plsc

N = 10000
NP_ = 10240
NP2 = NP_ + 8
E = 160000
D = 256
H = 128
NS = 16
U = 625
CH = 25
STRIPE = NP_ // NS
IZU = STRIPE // 16

_mesh = plsc.VectorSubcoreMesh(core_axis_name="c", subcore_axis_name="s")


def _spin(n):
    def body(i, carry):
        return carry + 1
    return lax.fori_loop(0, n, body, 0)


def _build_identity_units(iz_v, row0, lane):
    for k in range(IZU):
        iz_v[k, pl.ds(0, 16)] = row0 + 16 * k + lane
        for j in range(1, 8):
            iz_v[k, pl.ds(16 * j, 16)] = jnp.full((16,), NP_, jnp.int32)


@functools.partial(
    pl.kernel,
    out_type=jax.ShapeDtypeStruct((NP_, 16), jnp.float32),
    mesh=_mesh,
    scratch_types=[
        pltpu.VMEM((CH, 128), jnp.int32),
        pltpu.VMEM((IZU, 128), jnp.int32),
        pltpu.VMEM((128, 16), jnp.float32),
        pltpu.VMEM_SHARED((NP2, 16), jnp.float32),
    ],
)
def _deg_kernel(dst_hbm, deg_hbm, dst_v, iz_v, work_v, acc):
    c = lax.axis_index("c")
    s = lax.axis_index("s")
    row0 = pl.multiple_of(s * STRIPE, 8)
    lane = jax.lax.iota(jnp.int32, 16)
    _build_identity_units(iz_v, row0, lane)

    def wz(i, carry):
        work_v[i] = jnp.zeros((16,), jnp.float32)
        return carry
    lax.fori_loop(0, 128, wz, 0)

    for k in range(IZU):
        pltpu.sync_copy(work_v, acc.at[iz_v.at[k]])

    def wo(i, carry):
        work_v[i] = jnp.full((16,), 1.0, jnp.float32)
        return carry
    lax.fori_loop(0, 128, wo, 0)
    _spin(4000)
    plsc.subcore_barrier()

    def chunk(ci, carry):
        pltpu.sync_copy(dst_hbm.at[s, ci], dst_v)
        for k in range(CH):
            pltpu.sync_copy(work_v, acc.at[dst_v.at[k]], add=True)
        return carry
    lax.fori_loop(0, U // CH, chunk, 0)
    _spin(4000)
    plsc.subcore_barrier()

    @pl.when(c == 0)
    def _():
        pltpu.sync_copy(acc.at[pl.ds(row0, STRIPE)], deg_hbm.at[pl.ds(row0, STRIPE)])


@functools.partial(
    pl.kernel,
    out_type=jax.ShapeDtypeStruct((2 * NP_, H), jnp.float32),
    mesh=_mesh,
    scratch_types=[
        pltpu.VMEM((CH, 128), jnp.int32),
        pltpu.VMEM((CH, 128), jnp.int32),
        pltpu.VMEM((IZU, 128), jnp.int32),
        pltpu.VMEM((128, H), jnp.float32),
        pltpu.VMEM_SHARED((NP2, H), jnp.float32),
        pltpu.SemaphoreType.DMA,
    ],
)
def _spmm_kernel(src_hbm, dst_hbm, xs_hbm, out_hbm, src_v, dst_v, iz_v, gbuf,
                 acc, sem):
    c = lax.axis_index("c")
    s = lax.axis_index("s")
    row0 = pl.multiple_of(s * STRIPE, 8)
    base = pl.multiple_of(c * NP2 + row0, 8)
    lane = jax.lax.iota(jnp.int32, 16)
    _build_identity_units(iz_v, row0, lane)

    for k in range(IZU):
        pltpu.sync_copy(xs_hbm.at[pl.ds(base + 16 * k, 16)], gbuf.at[pl.ds(0, 16)])
        pltpu.sync_copy(gbuf, acc.at[iz_v.at[k]])
    _spin(4000)
    plsc.subcore_barrier()

    def chunk(ci, carry):
        pltpu.sync_copy(src_hbm.at[c * NS + s, ci], src_v)
        pltpu.sync_copy(dst_hbm.at[s, ci], dst_v)
        for k in range(CH):
            pltpu.async_copy(xs_hbm.at[src_v.at[k]], gbuf, sem).wait()
            pltpu.sync_copy(gbuf, acc.at[dst_v.at[k]], add=True)
        return carry
    lax.fori_loop(0, U // CH, chunk, 0)
    _spin(4000)
    plsc.subcore_barrier()

    pltpu.sync_copy(acc.at[pl.ds(row0, STRIPE)],
                    out_hbm.at[pl.ds(c * NP_ + row0, STRIPE)])


RB = 1000


def _prep_body(deg_ref, data_ref, w1_ref, out_ref):
    dinv = lax.rsqrt(deg_ref[:, 0:1] + 1.0)
    xw = jnp.dot(data_ref[...], w1_ref[...], preferred_element_type=jnp.float32)
    xs = xw * dinv
    out_ref[0] = xs[:, :H]
    out_ref[1] = xs[:, H:]


def _prep(deg_raw, data, w1):
    return pl.pallas_call(
        _prep_body,
        grid=(N // RB,),
        in_specs=[
            pl.BlockSpec((RB, 16), lambda i: (i, 0)),
            pl.BlockSpec((RB, D), lambda i: (i, 0)),
            pl.BlockSpec((D, D), lambda i: (0, 0)),
        ],
        out_specs=pl.BlockSpec((2, RB, H), lambda i: (0, i, 0)),
        out_shape=jax.ShapeDtypeStruct((2, NP2, H), jnp.float32),
    )(deg_raw, data, w1)


def _bn_prelu(x, g, be, a):
    m = jnp.mean(x, axis=0, keepdims=True)
    v = jnp.mean(x * x, axis=0, keepdims=True) - m * m
    y = g * (x - m) * lax.rsqrt(v + 1e-5) + be
    return jnp.where(y >= 0.0, y, a * y)


def _mid_body(scat_ref, deg_ref, b_ref, g_ref, be_ref, a_ref, w2_ref, out_ref):
    dinv = lax.rsqrt(deg_ref[:N, 0:1] + 1.0)
    a = a_ref[0, 0]
    hs = []
    for ci in range(2):
        sl = slice(ci * H, (ci + 1) * H)
        x = scat_ref[ci, :N, :] * dinv + b_ref[:, sl]
        hs.append(_bn_prelu(x, g_ref[:, sl], be_ref[:, sl], a))
    for ci in range(2):
        sl = slice(ci * H, (ci + 1) * H)
        z = (jnp.dot(hs[0], w2_ref[:H, sl], preferred_element_type=jnp.float32)
             + jnp.dot(hs[1], w2_ref[H:, sl], preferred_element_type=jnp.float32))
        out_ref[ci, :N, :] = z * dinv


def _mid(scat, deg_raw, b1, g1, be1, a1, w2):
    return pl.pallas_call(
        _mid_body,
        out_shape=jax.ShapeDtypeStruct((2, NP2, H), jnp.float32),
    )(scat, deg_raw, b1, g1, be1, a1, w2)


def _fin_body(scat_ref, deg_ref, b_ref, g_ref, be_ref, a_ref, out_ref):
    dinv = lax.rsqrt(deg_ref[:N, 0:1] + 1.0)
    a = a_ref[0, 0]
    for ci in range(2):
        sl = slice(ci * H, (ci + 1) * H)
        x = scat_ref[ci, :N, :] * dinv + b_ref[:, sl]
        out_ref[:, sl] = _bn_prelu(x, g_ref[:, sl], be_ref[:, sl], a)


def _fin(scat, deg_raw, b2, g2, be2, a2):
    return pl.pallas_call(
        _fin_body,
        out_shape=jax.ShapeDtypeStruct((N, D), jnp.float32),
    )(scat, deg_raw, b2, g2, be2, a2)


def kernel(data, edge_index, W1, b1, g1, be1, a1, W2, b2, g2, be2, a2):
    src = edge_index[0].reshape(NS, U, 16)
    dst = edge_index[1].reshape(NS, U, 16)
    dump_d = jnp.full((NS, U, 112), NP_, jnp.int32)
    dump_s = jnp.zeros((NS, U, 112), jnp.int32)
    dst_u = jnp.concatenate([dst, dump_d], axis=-1).reshape(NS, U // CH, CH, 128)
    src_p = jnp.concatenate([src, dump_s], axis=-1)
    src_u = jnp.stack([src_p, src_p + NP2]).reshape(2 * NS, U // CH, CH, 128)

    deg_raw = _deg_kernel(dst_u)
    deg_pad = jnp.zeros((NP2 - NP_, 16), jnp.float32)
    deg2 = jnp.concatenate([deg_raw, deg_pad])
    xs1 = _prep(deg2, data, W1)
    scat1 = _spmm_kernel(src_u, dst_u, xs1.reshape(2 * NP2, H))
    xs2 = _mid(scat1.reshape(2, NP_, H)[:, :N].reshape(2, N, H), deg2,
               b1.reshape(1, D), g1.reshape(1, D), be1.reshape(1, D),
               a1.reshape(1, 1), W2)
    scat2 = _spmm_kernel(src_u, dst_u, xs2.reshape(2 * NP2, H))
    return _fin(scat2.reshape(2, NP_, H)[:, :N].reshape(2, N, H), deg2,
                b2.reshape(1, D), g2.reshape(1, D), be2.reshape(1, D),
                a2.reshape(1, 1))
